# baseline (speedup 1.0000x reference)
import functools
import math

import jax
import jax.numpy as jnp
from jax import lax
from jax.experimental import pallas as pl
from jax.experimental.pallas import tpu as pltpu
from jax.experimental.pallas import tpu_sc as plsc

_K = 20
_NEG_INF = float("-inf")
_BIG_I32 = 2**31 - 1

_R = 128
_N = 100000
_BLK = 4096
_NB = math.ceil(_N / _BLK)
_SEG = 128
_SPB = _BLK // _SEG
_NT = _N // _SEG
_TAIL = _NT * _SEG
_NTAIL = _N - _TAIL
_CAP = 128
_WL = _NT + 48
_GT = 48

_NB1 = 12
_NB2 = _NB - _NB1
_NSEG1 = _NB1 * _SPB
_NSEG2 = _NT - _NSEG1
_S0_2 = _NSEG1


def _phase_a_body(l_ref, t_ref, bcep_ref, segmax_ref, *, off):
    j = pl.program_id(0)

    l = l_ref[...]
    t = t_ref[...]
    col = (j + off) * _BLK + lax.broadcasted_iota(jnp.int32, (_R, _BLK), 1)
    valid = col < _N

    bce = jnp.maximum(l, 0.0) - l * t + jnp.log1p(jnp.exp(-jnp.abs(l)))
    bce = jnp.where(valid, bce, 0.0)
    bcep_ref[...] = jnp.transpose(jnp.sum(bce, axis=1, keepdims=True))[None]

    lv = jnp.where(valid, l, _NEG_INF)
    segs = [
        jnp.max(lv[:, k * _SEG:(k + 1) * _SEG], axis=1, keepdims=True)
        for k in range(_SPB)
    ]
    segmax_ref[...] = jnp.concatenate(segs, axis=1)[None, :, :]


def _phase_a(logits, targets, off, nb):
    return pl.pallas_call(
        functools.partial(_phase_a_body, off=off),
        grid=(nb,),
        in_specs=[
            pl.BlockSpec((_R, _BLK), lambda j: (0, j + off)),
            pl.BlockSpec((_R, _BLK), lambda j: (0, j + off)),
        ],
        out_specs=[
            pl.BlockSpec((1, 1, _R), lambda j: (j, 0, 0)),
            pl.BlockSpec((1, _R, _SPB), lambda j: (j, 0, 0)),
        ],
        out_shape=[
            jax.ShapeDtypeStruct((nb, 1, _R), jnp.float32),
            jax.ShapeDtypeStruct((nb, _R, _SPB), jnp.float32),
        ],
        compiler_params=pltpu.CompilerParams(
            dimension_semantics=("parallel",),
        ),
    )(logits, targets)


def _phase_a2_body(segmax_ref, tau_ref):
    ids = lax.broadcasted_iota(jnp.int32, (_R, _NSEG1), 1)
    buf = segmax_ref[...]
    m = None
    for _ in range(_K):
        m = jnp.max(buf, axis=1, keepdims=True)
        ci = jnp.where(buf == m, ids, _BIG_I32)
        si = jnp.min(ci, axis=1, keepdims=True)
        buf = jnp.where(ids == si, _NEG_INF, buf)
    tau_ref[...] = m


def _phase_a2(segmax_t):
    return pl.pallas_call(
        _phase_a2_body,
        in_specs=[pl.BlockSpec((_R, _NSEG1), lambda: (0, 0))],
        out_specs=pl.BlockSpec((_R, 1), lambda: (0, 0)),
        out_shape=jax.ShapeDtypeStruct((_R, 1), jnp.float32),
    )(segmax_t)


def _phase_b_body(l_ref, t_ref, tau_ref, segmax_ref,
                  cv_ref, ct_ref, ci_ref,
                  tau_v, sm4, wl_v, stage_l, stage_t,
                  cv_v, ct_v, ci_v, sem_l, sem_t, sem_s,
                  *, s0, nb, nt_local):
    info = plsc.get_sparse_core_info()
    nc = info.num_cores
    wid = lax.axis_index("s") * nc + lax.axis_index("c")
    rows_per_w = _R // (nc * info.num_subcores)

    pltpu.sync_copy(tau_ref, tau_v.at[pl.ds(0, _R)])
    lane = lax.iota(jnp.int32, 16)

    for j in range(nb):
        pltpu.make_async_copy(
            segmax_ref.at[pl.ds(j * (_R * _SPB) + wid * (rows_per_w * _SPB),
                                rows_per_w * _SPB)],
            sm4.at[pl.ds(j * (rows_per_w * _SPB), rows_per_w * _SPB)],
            sem_s).start()
    for j in range(nb):
        pltpu.make_async_copy(
            segmax_ref.at[pl.ds(0, rows_per_w * _SPB)],
            sm4.at[pl.ds(0, rows_per_w * _SPB)], sem_s).wait()

    r8 = (wid // 2) * 8

    for k in range(rows_per_w):
        r = wid * rows_per_w + k
        rr = (wid % 2) * 4 + k
        tau_vec = jnp.full((16,), tau_v[pl.ds(r, 16)][0], jnp.float32)

        for i in range(_CAP // 16):
            cv_v[pl.ds(i * 16, 16)] = jnp.full((16,), _NEG_INF, jnp.float32)
            ct_v[pl.ds(i * 16, 16)] = jnp.zeros((16,), jnp.float32)
            ci_v[pl.ds(i * 16, 16)] = jnp.full((16,), _BIG_I32, jnp.int32)

        wptr = jnp.int32(0)
        for j in range(nb):
            for h in range(_SPB // 16):
                base = j * _SPB + h * 16
                if base >= nt_local:
                    continue
                m16 = sm4[pl.ds(j * (rows_per_w * _SPB) + k * _SPB + h * 16,
                                16)]
                act = m16 >= tau_vec
                if base + 16 > nt_local:
                    act = act & (lane < nt_local - base)
                s16 = s0 + base + lane
                pos = plsc.cumsum(act.astype(jnp.int32))
                widx = jnp.where(act, wptr + pos - 1, _NT + 16 + lane)
                plsc.store_scatter(wl_v, [widx], s16)
                wptr = wptr + pos[15]

        ngroups = (wptr + (_GT - 1)) // _GT

        def group_body(g, ptr):
            g0 = g * _GT
            gk = jnp.minimum(wptr - g0, _GT)

            def issue(i, _):
                s = wl_v[pl.ds(g0 + i, 16)][0]
                pltpu.make_async_copy(
                    l_ref.at[pl.ds(r8, 8), pl.ds(s * _SEG, _SEG)],
                    stage_l.at[i], sem_l).start()
                pltpu.make_async_copy(
                    t_ref.at[pl.ds(r8, 8), pl.ds(s * _SEG, _SEG)],
                    stage_t.at[i], sem_t).start()
                return jnp.int32(0)

            lax.fori_loop(0, gk, issue, jnp.int32(0))

            def drain(i, _):
                pltpu.make_async_copy(
                    l_ref.at[pl.ds(0, 8), pl.ds(0, _SEG)],
                    stage_l.at[0], sem_l).wait()
                pltpu.make_async_copy(
                    t_ref.at[pl.ds(0, 8), pl.ds(0, _SEG)],
                    stage_t.at[0], sem_t).wait()
                return jnp.int32(0)

            lax.fori_loop(0, gk, drain, jnp.int32(0))

            def proc(i, p):
                s = wl_v[pl.ds(g0 + i, 16)][0]
                cb = s * _SEG
                for v in range(_SEG // 16):
                    lv = stage_l[i, rr, pl.ds(v * 16, 16)]
                    gi = cb + v * 16 + lane
                    msk = lv >= tau_vec
                    cnt = plsc.all_reduce_population_count(msk)[0]

                    def emit(pp):
                        tv = stage_t[i, rr, pl.ds(v * 16, 16)]
                        pos = plsc.cumsum(msk.astype(jnp.int32))
                        cidx = jnp.where(
                            msk,
                            jnp.minimum(pp + pos - 1, _CAP + 15),
                            _CAP + lane,
                        )
                        plsc.store_scatter(cv_v, [cidx], lv)
                        plsc.store_scatter(ct_v, [cidx], tv)
                        plsc.store_scatter(ci_v, [cidx], gi)
                        return pp + cnt

                    p = lax.cond(cnt > 0, emit, lambda pp: pp, p)
                return p

            return lax.fori_loop(0, gk, proc, ptr)

        lax.fori_loop(0, ngroups, group_body, jnp.int32(0))

        pltpu.sync_copy(cv_v.at[pl.ds(0, _CAP)],
                        cv_ref.at[pl.ds(r * _CAP, _CAP)])
        pltpu.sync_copy(ct_v.at[pl.ds(0, _CAP)],
                        ct_ref.at[pl.ds(r * _CAP, _CAP)])
        pltpu.sync_copy(ci_v.at[pl.ds(0, _CAP)],
                        ci_ref.at[pl.ds(r * _CAP, _CAP)])


def _phase_b(logits, targets, tau_flat, segmax_flat, s0, nb, nt_local):
    mesh = plsc.VectorSubcoreMesh(core_axis_name="c", subcore_axis_name="s")
    kb = functools.partial(
        pl.kernel,
        mesh=mesh,
        compiler_params=pltpu.CompilerParams(
            needs_layout_passes=False, use_tc_tiling_on_sc=True),
        out_type=[
            jax.ShapeDtypeStruct((_R * _CAP,), jnp.float32),
            jax.ShapeDtypeStruct((_R * _CAP,), jnp.float32),
            jax.ShapeDtypeStruct((_R * _CAP,), jnp.int32),
        ],
        scratch_types=[
            pltpu.VMEM((_R + 16,), jnp.float32),
            pltpu.VMEM((nb * 4 * _SPB,), jnp.float32),
            pltpu.VMEM((_WL,), jnp.int32),
            pltpu.VMEM((_GT, 8, _SEG), jnp.float32),
            pltpu.VMEM((_GT, 8, _SEG), jnp.float32),
            pltpu.VMEM((_CAP + 16,), jnp.float32),
            pltpu.VMEM((_CAP + 16,), jnp.float32),
            pltpu.VMEM((_CAP + 16,), jnp.int32),
            pltpu.SemaphoreType.DMA,
            pltpu.SemaphoreType.DMA,
            pltpu.SemaphoreType.DMA,
        ],
    )(functools.partial(_phase_b_body, s0=s0, nb=nb, nt_local=nt_local))
    return kb(logits, targets, tau_flat, segmax_flat)


_CTOT = 2 * _CAP + _NTAIL


def _phase_c_body(cv1_ref, ct1_ref, ci1_ref, cv2_ref, ct2_ref, ci2_ref,
                  tl_ref, tt_ref, bcep1_ref, bcep2_ref, bcel_ref, out_ref):
    tail_ids = _TAIL + lax.broadcasted_iota(jnp.int32, (_R, _NTAIL), 1)
    buf = jnp.concatenate([cv1_ref[...], cv2_ref[...], tl_ref[...]], axis=1)
    ibuf = jnp.concatenate([ci1_ref[...], ci2_ref[...], tail_ids], axis=1)
    tbuf = jnp.concatenate([ct1_ref[...], ct2_ref[...], tt_ref[...]], axis=1)

    picks_v, picks_t = [], []
    for _ in range(_K):
        m = jnp.max(buf, axis=1, keepdims=True)
        ci = jnp.where(buf == m, ibuf, _BIG_I32)
        si = jnp.min(ci, axis=1, keepdims=True)
        sel = ibuf == si
        tm = jnp.sum(jnp.where(sel, tbuf, 0.0), axis=1, keepdims=True)
        picks_v.append(m)
        picks_t.append(tm)
        buf = jnp.where(sel, _NEG_INF, buf)

    vs = jnp.concatenate(picks_v, axis=1)
    ts = jnp.concatenate(picks_t, axis=1)
    probs = jax.nn.sigmoid(vs)
    logp = jnp.maximum(jnp.log(probs), -100.0)
    log1mp = jnp.maximum(jnp.log(1.0 - probs), -100.0)
    row_bce = -jnp.sum(ts * logp + (1.0 - ts) * log1mp, axis=1) / _K
    top_loss = jnp.sum(row_bce) / _R
    bce_sum = jnp.sum(bcep1_ref[...]) + jnp.sum(bcep2_ref[...])
    bce_mean = bce_sum / (_R * _N)
    out_ref[0, 0] = bce_mean + bcel_ref[0, 0] * top_loss


def _phase_c(cv1, ct1, ci1, cv2, ct2, ci2, tail_l, tail_t,
             bcep1, bcep2, bcel):
    return pl.pallas_call(
        _phase_c_body,
        in_specs=[
            pl.BlockSpec((_R, _CAP), lambda: (0, 0)),
            pl.BlockSpec((_R, _CAP), lambda: (0, 0)),
            pl.BlockSpec((_R, _CAP), lambda: (0, 0)),
            pl.BlockSpec((_R, _CAP), lambda: (0, 0)),
            pl.BlockSpec((_R, _CAP), lambda: (0, 0)),
            pl.BlockSpec((_R, _CAP), lambda: (0, 0)),
            pl.BlockSpec((_R, _NTAIL), lambda: (0, 0)),
            pl.BlockSpec((_R, _NTAIL), lambda: (0, 0)),
            pl.BlockSpec((_NB1, 1, _R), lambda: (0, 0, 0)),
            pl.BlockSpec((_NB2, 1, _R), lambda: (0, 0, 0)),
            pl.BlockSpec(memory_space=pltpu.SMEM),
        ],
        out_specs=pl.BlockSpec(memory_space=pltpu.SMEM),
        out_shape=jax.ShapeDtypeStruct((1, 1), jnp.float32),
    )(cv1, ct1, ci1, cv2, ct2, ci2, tail_l, tail_t, bcep1, bcep2, bcel)


@jax.jit
def kernel(logits, targets, BCE_L):
    bcel = jnp.reshape(BCE_L, (1, 1)).astype(jnp.float32)

    bcep1, segmax1 = _phase_a(logits, targets, 0, _NB1)
    smt1 = jnp.reshape(jnp.transpose(segmax1, (1, 0, 2)), (_R, _NSEG1))
    tau = _phase_a2(smt1)
    tau_flat = jnp.reshape(tau, (-1,))

    cv1, ct1, ci1 = _phase_b(
        logits, targets, tau_flat, jnp.reshape(segmax1, (-1,)),
        s0=0, nb=_NB1, nt_local=_NSEG1)

    bcep2, segmax2 = _phase_a(logits, targets, _NB1, _NB2)

    cv2, ct2, ci2 = _phase_b(
        logits, targets, tau_flat, jnp.reshape(segmax2, (-1,)),
        s0=_S0_2, nb=_NB2, nt_local=_NSEG2)

    tail_l = lax.slice(logits, (0, _TAIL), (_R, _N))
    tail_t = lax.slice(targets, (0, _TAIL), (_R, _N))

    out = _phase_c(
        jnp.reshape(cv1, (_R, _CAP)),
        jnp.reshape(ct1, (_R, _CAP)),
        jnp.reshape(ci1, (_R, _CAP)),
        jnp.reshape(cv2, (_R, _CAP)),
        jnp.reshape(ct2, (_R, _CAP)),
        jnp.reshape(ci2, (_R, _CAP)),
        tail_l,
        tail_t,
        bcep1,
        bcep2,
        bcel,
    )
    return out[0, 0]


# speedup vs baseline: 1.0795x; 1.0795x over previous
import functools
import math

import jax
import jax.numpy as jnp
from jax import lax
from jax.experimental import pallas as pl
from jax.experimental.pallas import tpu as pltpu
from jax.experimental.pallas import tpu_sc as plsc

_K = 20
_NEG_INF = float("-inf")
_BIG_I32 = 2**31 - 1

_R = 128
_N = 100000
_BLK = 4096
_NB = math.ceil(_N / _BLK)
_SEG = 128
_SPB = _BLK // _SEG
_NT = _N // _SEG
_TAIL = _NT * _SEG
_NTAIL = _N - _TAIL
_CAP = 128
_WL = _NT + 48
_GT = 48

_NSEGPAD = _NB * _SPB


def _phase_a_body(l_ref, t_ref, bcep_ref, segmax_ref, *, off):
    j = pl.program_id(0)

    l = l_ref[...]
    t = t_ref[...]
    col = (j + off) * _BLK + lax.broadcasted_iota(jnp.int32, (_R, _BLK), 1)
    valid = col < _N

    bce = jnp.maximum(l, 0.0) - l * t + jnp.log1p(jnp.exp(-jnp.abs(l)))
    bce = jnp.where(valid, bce, 0.0)
    bcep_ref[...] = jnp.transpose(jnp.sum(bce, axis=1, keepdims=True))[None]

    lv = jnp.where(valid, l, _NEG_INF)
    segs = [
        jnp.max(lv[:, k * _SEG:(k + 1) * _SEG], axis=1, keepdims=True)
        for k in range(_SPB)
    ]
    segmax_ref[...] = jnp.concatenate(segs, axis=1)[None, :, :]


def _phase_a(logits, targets, off, nb):
    return pl.pallas_call(
        functools.partial(_phase_a_body, off=off),
        grid=(nb,),
        in_specs=[
            pl.BlockSpec((_R, _BLK), lambda j: (0, j + off)),
            pl.BlockSpec((_R, _BLK), lambda j: (0, j + off)),
        ],
        out_specs=[
            pl.BlockSpec((1, 1, _R), lambda j: (j, 0, 0)),
            pl.BlockSpec((1, _R, _SPB), lambda j: (j, 0, 0)),
        ],
        out_shape=[
            jax.ShapeDtypeStruct((nb, 1, _R), jnp.float32),
            jax.ShapeDtypeStruct((nb, _R, _SPB), jnp.float32),
        ],
        compiler_params=pltpu.CompilerParams(
            dimension_semantics=("parallel",),
        ),
    )(logits, targets)


def _phase_a2_body(segmax_ref, tau_ref):
    ids = lax.broadcasted_iota(jnp.int32, (_R, _NSEGPAD), 1)
    buf = segmax_ref[...]
    m = None
    for _ in range(_K):
        m = jnp.max(buf, axis=1, keepdims=True)
        ci = jnp.where(buf == m, ids, _BIG_I32)
        si = jnp.min(ci, axis=1, keepdims=True)
        buf = jnp.where(ids == si, _NEG_INF, buf)
    tau_ref[...] = m


def _phase_a2(segmax_t):
    return pl.pallas_call(
        _phase_a2_body,
        in_specs=[pl.BlockSpec((_R, _NSEGPAD), lambda: (0, 0))],
        out_specs=pl.BlockSpec((_R, 1), lambda: (0, 0)),
        out_shape=jax.ShapeDtypeStruct((_R, 1), jnp.float32),
    )(segmax_t)


def _phase_b_body(l_ref, t_ref, tau_ref, segmax_ref,
                  cv_ref, ct_ref, ci_ref,
                  tau_v, sm4, wl_v, stage_l, stage_t,
                  cv_v, ct_v, ci_v, sem_l, sem_t, sem_s,
                  *, s0, nb, nt_local):
    info = plsc.get_sparse_core_info()
    nc = info.num_cores
    wid = lax.axis_index("s") * nc + lax.axis_index("c")
    rows_per_w = _R // (nc * info.num_subcores)

    pltpu.sync_copy(tau_ref, tau_v.at[pl.ds(0, _R)])
    lane = lax.iota(jnp.int32, 16)

    for j in range(nb):
        pltpu.make_async_copy(
            segmax_ref.at[pl.ds(j * (_R * _SPB) + wid * (rows_per_w * _SPB),
                                rows_per_w * _SPB)],
            sm4.at[pl.ds(j * (rows_per_w * _SPB), rows_per_w * _SPB)],
            sem_s).start()
    for j in range(nb):
        pltpu.make_async_copy(
            segmax_ref.at[pl.ds(0, rows_per_w * _SPB)],
            sm4.at[pl.ds(0, rows_per_w * _SPB)], sem_s).wait()

    r8 = (wid // 2) * 8

    for k in range(rows_per_w):
        r = wid * rows_per_w + k
        rr = (wid % 2) * 4 + k
        tau_vec = jnp.full((16,), tau_v[pl.ds(r, 16)][0], jnp.float32)

        for i in range(_CAP // 16):
            cv_v[pl.ds(i * 16, 16)] = jnp.full((16,), _NEG_INF, jnp.float32)
            ct_v[pl.ds(i * 16, 16)] = jnp.zeros((16,), jnp.float32)
            ci_v[pl.ds(i * 16, 16)] = jnp.full((16,), _BIG_I32, jnp.int32)

        wptr = jnp.int32(0)
        for j in range(nb):
            for h in range(_SPB // 16):
                base = j * _SPB + h * 16
                if base >= nt_local:
                    continue
                m16 = sm4[pl.ds(j * (rows_per_w * _SPB) + k * _SPB + h * 16,
                                16)]
                act = m16 >= tau_vec
                if base + 16 > nt_local:
                    act = act & (lane < nt_local - base)
                s16 = s0 + base + lane
                pos = plsc.cumsum(act.astype(jnp.int32))
                widx = jnp.where(act, wptr + pos - 1, _NT + 16 + lane)
                plsc.store_scatter(wl_v, [widx], s16)
                wptr = wptr + pos[15]

        ngroups = (wptr + (_GT - 1)) // _GT

        def group_body(g, ptr):
            g0 = g * _GT
            gk = jnp.minimum(wptr - g0, _GT)

            def issue(i, _):
                s = wl_v[pl.ds(g0 + i, 16)][0]
                pltpu.make_async_copy(
                    l_ref.at[pl.ds(r8, 8), pl.ds(s * _SEG, _SEG)],
                    stage_l.at[i], sem_l).start()
                pltpu.make_async_copy(
                    t_ref.at[pl.ds(r8, 8), pl.ds(s * _SEG, _SEG)],
                    stage_t.at[i], sem_t).start()
                return jnp.int32(0)

            lax.fori_loop(0, gk, issue, jnp.int32(0))

            def drain(i, _):
                pltpu.make_async_copy(
                    l_ref.at[pl.ds(0, 8), pl.ds(0, _SEG)],
                    stage_l.at[0], sem_l).wait()
                pltpu.make_async_copy(
                    t_ref.at[pl.ds(0, 8), pl.ds(0, _SEG)],
                    stage_t.at[0], sem_t).wait()
                return jnp.int32(0)

            lax.fori_loop(0, gk, drain, jnp.int32(0))

            def proc(i, p):
                s = wl_v[pl.ds(g0 + i, 16)][0]
                cb = s * _SEG
                for v in range(_SEG // 16):
                    lv = stage_l[i, rr, pl.ds(v * 16, 16)]
                    gi = cb + v * 16 + lane
                    msk = lv >= tau_vec
                    cnt = plsc.all_reduce_population_count(msk)[0]

                    def emit(pp):
                        tv = stage_t[i, rr, pl.ds(v * 16, 16)]
                        pos = plsc.cumsum(msk.astype(jnp.int32))
                        cidx = jnp.where(
                            msk,
                            jnp.minimum(pp + pos - 1, _CAP + 15),
                            _CAP + lane,
                        )
                        plsc.store_scatter(cv_v, [cidx], lv)
                        plsc.store_scatter(ct_v, [cidx], tv)
                        plsc.store_scatter(ci_v, [cidx], gi)
                        return pp + cnt

                    p = lax.cond(cnt > 0, emit, lambda pp: pp, p)
                return p

            return lax.fori_loop(0, gk, proc, ptr)

        lax.fori_loop(0, ngroups, group_body, jnp.int32(0))

        pltpu.sync_copy(cv_v.at[pl.ds(0, _CAP)],
                        cv_ref.at[pl.ds(r * _CAP, _CAP)])
        pltpu.sync_copy(ct_v.at[pl.ds(0, _CAP)],
                        ct_ref.at[pl.ds(r * _CAP, _CAP)])
        pltpu.sync_copy(ci_v.at[pl.ds(0, _CAP)],
                        ci_ref.at[pl.ds(r * _CAP, _CAP)])


def _phase_b(logits, targets, tau_flat, segmax_flat, s0, nb, nt_local):
    mesh = plsc.VectorSubcoreMesh(core_axis_name="c", subcore_axis_name="s")
    kb = functools.partial(
        pl.kernel,
        mesh=mesh,
        compiler_params=pltpu.CompilerParams(
            needs_layout_passes=False, use_tc_tiling_on_sc=True),
        out_type=[
            jax.ShapeDtypeStruct((_R * _CAP,), jnp.float32),
            jax.ShapeDtypeStruct((_R * _CAP,), jnp.float32),
            jax.ShapeDtypeStruct((_R * _CAP,), jnp.int32),
        ],
        scratch_types=[
            pltpu.VMEM((_R + 16,), jnp.float32),
            pltpu.VMEM((nb * 4 * _SPB,), jnp.float32),
            pltpu.VMEM((_WL,), jnp.int32),
            pltpu.VMEM((_GT, 8, _SEG), jnp.float32),
            pltpu.VMEM((_GT, 8, _SEG), jnp.float32),
            pltpu.VMEM((_CAP + 16,), jnp.float32),
            pltpu.VMEM((_CAP + 16,), jnp.float32),
            pltpu.VMEM((_CAP + 16,), jnp.int32),
            pltpu.SemaphoreType.DMA,
            pltpu.SemaphoreType.DMA,
            pltpu.SemaphoreType.DMA,
        ],
    )(functools.partial(_phase_b_body, s0=s0, nb=nb, nt_local=nt_local))
    return kb(logits, targets, tau_flat, segmax_flat)


def _phase_c_body(cv_ref, ct_ref, ci_ref,
                  tl_ref, tt_ref, bcep_ref, bcel_ref, out_ref):
    tail_ids = _TAIL + lax.broadcasted_iota(jnp.int32, (_R, _NTAIL), 1)
    buf = jnp.concatenate([cv_ref[...], tl_ref[...]], axis=1)
    ibuf = jnp.concatenate([ci_ref[...], tail_ids], axis=1)
    tbuf = jnp.concatenate([ct_ref[...], tt_ref[...]], axis=1)

    picks_v, picks_t = [], []
    for _ in range(_K):
        m = jnp.max(buf, axis=1, keepdims=True)
        ci = jnp.where(buf == m, ibuf, _BIG_I32)
        si = jnp.min(ci, axis=1, keepdims=True)
        sel = ibuf == si
        tm = jnp.sum(jnp.where(sel, tbuf, 0.0), axis=1, keepdims=True)
        picks_v.append(m)
        picks_t.append(tm)
        buf = jnp.where(sel, _NEG_INF, buf)

    vs = jnp.concatenate(picks_v, axis=1)
    ts = jnp.concatenate(picks_t, axis=1)
    probs = jax.nn.sigmoid(vs)
    logp = jnp.maximum(jnp.log(probs), -100.0)
    log1mp = jnp.maximum(jnp.log(1.0 - probs), -100.0)
    row_bce = -jnp.sum(ts * logp + (1.0 - ts) * log1mp, axis=1) / _K
    top_loss = jnp.sum(row_bce) / _R
    bce_mean = jnp.sum(bcep_ref[...]) / (_R * _N)
    out_ref[0, 0] = bce_mean + bcel_ref[0, 0] * top_loss


def _phase_c(cv, ct, ci, tail_l, tail_t, bcep, bcel):
    return pl.pallas_call(
        _phase_c_body,
        in_specs=[
            pl.BlockSpec((_R, _CAP), lambda: (0, 0)),
            pl.BlockSpec((_R, _CAP), lambda: (0, 0)),
            pl.BlockSpec((_R, _CAP), lambda: (0, 0)),
            pl.BlockSpec((_R, _NTAIL), lambda: (0, 0)),
            pl.BlockSpec((_R, _NTAIL), lambda: (0, 0)),
            pl.BlockSpec((_NB, 1, _R), lambda: (0, 0, 0)),
            pl.BlockSpec(memory_space=pltpu.SMEM),
        ],
        out_specs=pl.BlockSpec(memory_space=pltpu.SMEM),
        out_shape=jax.ShapeDtypeStruct((1, 1), jnp.float32),
    )(cv, ct, ci, tail_l, tail_t, bcep, bcel)


@jax.jit
def kernel(logits, targets, BCE_L):
    bcel = jnp.reshape(BCE_L, (1, 1)).astype(jnp.float32)

    bcep, segmax = _phase_a(logits, targets, 0, _NB)
    smt = jnp.reshape(jnp.transpose(segmax, (1, 0, 2)), (_R, _NSEGPAD))
    tau = _phase_a2(smt)

    cv, ct, ci = _phase_b(
        logits, targets, jnp.reshape(tau, (-1,)),
        jnp.reshape(segmax, (-1,)),
        s0=0, nb=_NB, nt_local=_NT)

    tail_l = lax.slice(logits, (0, _TAIL), (_R, _N))
    tail_t = lax.slice(targets, (0, _TAIL), (_R, _N))

    out = _phase_c(
        jnp.reshape(cv, (_R, _CAP)),
        jnp.reshape(ct, (_R, _CAP)),
        jnp.reshape(ci, (_R, _CAP)),
        tail_l,
        tail_t,
        bcep,
        bcel,
    )
    return out[0, 0]
